# baseline (speedup 1.0000x reference)
import math

import jax
import jax.numpy as jnp
from jax import lax
from jax.experimental import pallas as pl
from jax.experimental.pallas import tpu as pltpu
from jax.experimental.pallas import tpu_sc as plsc

VOCAB = 1000000
D = 64
ROWS = 4096
COLS = 50
NC = 2
NS = 16
NW = NC * NS
SCALE = math.sqrt(D)

VB = 256
LANES = 128
NBLK = VOCAB // VB
BASE_BLK = NBLK // NW
EXTRA = NBLK - BASE_BLK * NW
HALF = BASE_BLK // 2
PAIRS = NBLK * (VB // 2) + D
assert VOCAB - NBLK * VB == D

SB = ROWS // NW
COLS_PAD = 56


def _table_body(emb_t, tail_p, table, in0, in1, out0, out1, tbuf,
                isem0, isem1, osem0, osem1):
    w = lax.axis_index("s") * NC + lax.axis_index("c")
    start = w * BASE_BLK + jnp.minimum(w, EXTRA)
    ins = (in0, in1)
    outs = (out0, out1)
    isems = (isem0, isem1)
    osems = (osem0, osem1)

    def src(b):
        return emb_t.at[:, pl.ds(pl.multiple_of(b * VB, VB), VB)]

    def dst(b):
        return table.at[pl.ds(pl.multiple_of(b * (VB // 2), VB // 2),
                              VB // 2), :]

    def transpose_scale(ib, ob):
        iot = lax.iota(jnp.int32, 16)
        for jc in range(D // 16):
            jv = iot + jc * 16

            @plsc.parallel_loop(0, (VB // 16) * 16, unroll=8)
            def _(i):
                vv = ((iot + i) & 15) + ((i >> 4) << 4)
                vals = plsc.load_gather(ib, [jv, vv])
                plsc.store_scatter(
                    ob, [((vv >> 1) & ~63) + (vv & 63), (vv & 64) + jv],
                    vals * SCALE)

    pltpu.async_copy(src(start), ins[0], isems[0])
    pltpu.async_copy(src(start + 1), ins[1], isems[1])

    def step(i, carry):
        for par in range(2):
            b = start + 2 * i + par
            pltpu.make_async_copy(src(b), ins[par], isems[par]).wait()

            @pl.when(i > 0)
            def _():
                pltpu.make_async_copy(
                    dst(b - 2), outs[par], osems[par]).wait()
            transpose_scale(ins[par], outs[par])
            pltpu.async_copy(outs[par], dst(b), osems[par])

            @pl.when(2 * i + par + 2 < BASE_BLK)
            def _():
                pltpu.async_copy(src(b + 2), ins[par], isems[par])
        return carry

    lax.fori_loop(0, HALF, step, 0)
    pltpu.make_async_copy(dst(start + BASE_BLK - 2), outs[0], osems[0]).wait()
    pltpu.make_async_copy(dst(start + BASE_BLK - 1), outs[1], osems[1]).wait()

    @pl.when(w < EXTRA)
    def _():
        b = start + BASE_BLK
        pltpu.async_copy(src(b), ins[0], isems[0]).wait()
        transpose_scale(ins[0], outs[0])
        pltpu.async_copy(outs[0], dst(b), osems[0]).wait()

    @pl.when(w == NW - 1)
    def _():
        pltpu.async_copy(tail_p, tbuf, isems[0]).wait()
        pltpu.async_copy(
            tbuf, table.at[pl.ds(NBLK * (VB // 2), VOCAB - NBLK * VB), :],
            osems[0]
        ).wait()


def _gather_body(table, xt, out, xv, xp, g0, g1, o0, o1,
                 gsem0, gsem1, osem0, osem1):
    w = lax.axis_index("s") * NC + lax.axis_index("c")
    soff = pl.multiple_of(w * SB, SB)
    gs = (g0, g1)
    os_ = (o0, o1)
    gsems = (gsem0, gsem1)
    osems = (osem0, osem1)
    iot = lax.iota(jnp.int32, 16)

    pltpu.sync_copy(xt.at[:, pl.ds(soff, SB)], xv)

    @plsc.parallel_loop(0, COLS_PAD * (SB // 16), unroll=8)
    def _(i):
        r = i >> 3
        c0 = (i & 7) * 16
        v = xv[r, pl.ds(c0, 16)]
        xp[r, pl.ds(c0, 16)] = ((v >> 1) & ~63) | (v & 63)

    def gather(t, par):
        return pltpu.async_copy(table.at[xp.at[t]], gs[par], gsems[par])

    def wait_gather(t, par):
        pltpu.make_async_copy(table.at[xp.at[t]], gs[par], gsems[par]).wait()

    def put(t, par):
        return pltpu.async_copy(
            os_[par], out.at[t, :, pl.ds(soff, SB)], osems[par])

    def wait_put(t, par):
        pltpu.make_async_copy(
            os_[par], out.at[t, :, pl.ds(soff, SB)], osems[par]).wait()

    def transpose(t, g, o):
        for sb in range(SB // 16):
            s0 = sb * 16
            parv = xv[t, pl.ds(s0, 16)] & 64

            @plsc.parallel_loop(0, (D // 16) * 16, unroll=8)
            def _(i):
                d = i & 15
                jbase = i >> 4
                sv = iot + s0
                jv = ((iot + d) & 15) + jbase * 16
                vals = plsc.load_gather(g, [sv, parv + jv])
                plsc.store_scatter(o, [jv, sv], vals)

    gather(0, 0)
    gather(1, 1)

    def step(i, carry):
        for par in range(2):
            t = 2 * i + par
            wait_gather(t, par)

            @pl.when(i > 0)
            def _():
                wait_put(t - 2, par)
            transpose(t, gs[par], os_[par])
            put(t, par)

            @pl.when(t + 2 < COLS)
            def _():
                gather(t + 2, par)
        return carry

    lax.fori_loop(0, COLS // 2, step, 0)
    wait_put(COLS - 2, 0)
    wait_put(COLS - 1, 1)


@jax.jit
def kernel(x, embedding):
    emb_t = embedding.T
    tail_p = jnp.pad(embedding[NBLK * VB:] * SCALE,
                     ((0, 0), (0, LANES - D)))
    xt = jnp.pad(x.astype(jnp.int32).T,
                 ((0, COLS_PAD - COLS), (0, 0)))

    k1 = pl.kernel(
        _table_body,
        out_type=jax.ShapeDtypeStruct((PAIRS, LANES), jnp.float32),
        mesh=plsc.VectorSubcoreMesh(core_axis_name="c", subcore_axis_name="s"),
        scratch_types=[
            pltpu.VMEM((D, VB), jnp.float32),
            pltpu.VMEM((D, VB), jnp.float32),
            pltpu.VMEM((VB // 2, LANES), jnp.float32),
            pltpu.VMEM((VB // 2, LANES), jnp.float32),
            pltpu.VMEM((VOCAB - NBLK * VB, LANES), jnp.float32),
            pltpu.SemaphoreType.DMA,
            pltpu.SemaphoreType.DMA,
            pltpu.SemaphoreType.DMA,
            pltpu.SemaphoreType.DMA,
        ],
        compiler_params=pltpu.CompilerParams(
            use_tc_tiling_on_sc=True, needs_layout_passes=False),
    )
    table = k1(emb_t, tail_p)

    k2 = pl.kernel(
        _gather_body,
        out_type=jax.ShapeDtypeStruct((COLS, D, ROWS), jnp.float32),
        mesh=plsc.VectorSubcoreMesh(core_axis_name="c", subcore_axis_name="s"),
        scratch_types=[
            pltpu.VMEM((COLS_PAD, SB), jnp.int32),
            pltpu.VMEM((COLS_PAD, SB), jnp.int32),
            pltpu.VMEM((SB, LANES), jnp.float32),
            pltpu.VMEM((SB, LANES), jnp.float32),
            pltpu.VMEM((D, SB), jnp.float32),
            pltpu.VMEM((D, SB), jnp.float32),
            pltpu.SemaphoreType.DMA,
            pltpu.SemaphoreType.DMA,
            pltpu.SemaphoreType.DMA,
            pltpu.SemaphoreType.DMA,
        ],
        compiler_params=pltpu.CompilerParams(
            use_tc_tiling_on_sc=True, needs_layout_passes=False),
    )
    out3 = k2(table, xt)
    return out3.transpose(2, 0, 1)


# speedup vs baseline: 1.0616x; 1.0616x over previous
import math

import jax
import jax.numpy as jnp
from jax import lax
from jax.experimental import pallas as pl
from jax.experimental.pallas import tpu as pltpu
from jax.experimental.pallas import tpu_sc as plsc

VOCAB = 1000000
D = 64
ROWS = 4096
COLS = 50
NC = 2
NS = 16
NW = NC * NS
SCALE = math.sqrt(D)

VB = 128
LANES = 128
NBLK = VOCAB // VB
BASE_BLK = NBLK // NW
EXTRA = NBLK - BASE_BLK * NW
HALF = BASE_BLK // 2
PAIRS = NBLK * (VB // 2) + D
assert VOCAB - NBLK * VB == D

SB = ROWS // NW
COLS_PAD = 56


def _table_body(emb_t, tail_p, table, in0, in1, out0, out1, tbuf,
                isem0, isem1, osem0, osem1):
    w = lax.axis_index("s") * NC + lax.axis_index("c")
    start = w * BASE_BLK + jnp.minimum(w, EXTRA)
    ins = (in0, in1)
    outs = (out0, out1)
    isems = (isem0, isem1)
    osems = (osem0, osem1)

    def src(b):
        return emb_t.at[:, pl.ds(pl.multiple_of(b * VB, VB), VB)]

    def dst(b):
        return table.at[pl.ds(pl.multiple_of(b * (VB // 2), VB // 2),
                              VB // 2), :]

    def transpose_scale(ib, ob):
        iot = lax.iota(jnp.int32, 16)
        for jc in range(D // 16):
            jv = iot + jc * 16

            @plsc.parallel_loop(0, (VB // 16) * 16, unroll=8)
            def _(i):
                vv = ((iot + i) & 15) + ((i >> 4) << 4)
                vals = plsc.load_gather(ib, [jv, vv])
                plsc.store_scatter(
                    ob, [vv & 63, (vv & 64) + jv], vals * SCALE)

    pltpu.async_copy(src(start), ins[0], isems[0])
    pltpu.async_copy(src(start + 1), ins[1], isems[1])

    def step(i, carry):
        for par in range(2):
            b = start + 2 * i + par
            pltpu.make_async_copy(src(b), ins[par], isems[par]).wait()

            @pl.when(i > 0)
            def _():
                pltpu.make_async_copy(
                    dst(b - 2), outs[par], osems[par]).wait()
            transpose_scale(ins[par], outs[par])
            pltpu.async_copy(outs[par], dst(b), osems[par])

            @pl.when(2 * i + par + 2 < BASE_BLK)
            def _():
                pltpu.async_copy(src(b + 2), ins[par], isems[par])
        return carry

    lax.fori_loop(0, HALF, step, 0)
    pltpu.make_async_copy(dst(start + BASE_BLK - 2), outs[0], osems[0]).wait()
    pltpu.make_async_copy(dst(start + BASE_BLK - 1), outs[1], osems[1]).wait()

    @pl.when(w < EXTRA)
    def _():
        b = start + BASE_BLK
        pltpu.async_copy(src(b), ins[0], isems[0]).wait()
        transpose_scale(ins[0], outs[0])
        pltpu.async_copy(outs[0], dst(b), osems[0]).wait()

    @pl.when(w == NW - 1)
    def _():
        pltpu.async_copy(tail_p, tbuf, isems[0]).wait()
        pltpu.async_copy(
            tbuf, table.at[pl.ds(NBLK * (VB // 2), VOCAB - NBLK * VB), :],
            osems[0]
        ).wait()


def _gather_body(table, xt, out, xv, xp, g0, g1, o0, o1,
                 gsem0, gsem1, osem0, osem1):
    w = lax.axis_index("s") * NC + lax.axis_index("c")
    soff = pl.multiple_of(w * SB, SB)
    gs = (g0, g1)
    os_ = (o0, o1)
    gsems = (gsem0, gsem1)
    osems = (osem0, osem1)
    iot = lax.iota(jnp.int32, 16)

    pltpu.sync_copy(xt.at[:, pl.ds(soff, SB)], xv)

    @plsc.parallel_loop(0, COLS_PAD * (SB // 16), unroll=8)
    def _(i):
        r = i >> 3
        c0 = (i & 7) * 16
        v = xv[r, pl.ds(c0, 16)]
        xp[r, pl.ds(c0, 16)] = ((v >> 1) & ~63) | (v & 63)

    def gather(t, par):
        return pltpu.async_copy(table.at[xp.at[t]], gs[par], gsems[par])

    def wait_gather(t, par):
        pltpu.make_async_copy(table.at[xp.at[t]], gs[par], gsems[par]).wait()

    def put(t, par):
        return pltpu.async_copy(
            os_[par], out.at[t, :, pl.ds(soff, SB)], osems[par])

    def wait_put(t, par):
        pltpu.make_async_copy(
            os_[par], out.at[t, :, pl.ds(soff, SB)], osems[par]).wait()

    def transpose(t, g, o):
        for sb in range(SB // 16):
            s0 = sb * 16
            parv = xv[t, pl.ds(s0, 16)] & 64

            @plsc.parallel_loop(0, (D // 16) * 16, unroll=8)
            def _(i):
                d = i & 15
                jbase = i >> 4
                sv = iot + s0
                jv = ((iot + d) & 15) + jbase * 16
                vals = plsc.load_gather(g, [sv, parv + jv])
                plsc.store_scatter(o, [jv, sv], vals)

    gather(0, 0)
    gather(1, 1)

    def step(i, carry):
        for par in range(2):
            t = 2 * i + par
            wait_gather(t, par)

            @pl.when(i > 0)
            def _():
                wait_put(t - 2, par)
            transpose(t, gs[par], os_[par])
            put(t, par)

            @pl.when(t + 2 < COLS)
            def _():
                gather(t + 2, par)
        return carry

    lax.fori_loop(0, COLS // 2, step, 0)
    wait_put(COLS - 2, 0)
    wait_put(COLS - 1, 1)


@jax.jit
def kernel(x, embedding):
    emb_t = embedding.T
    tail_p = jnp.pad(embedding[NBLK * VB:] * SCALE,
                     ((0, 0), (0, LANES - D)))
    xt = jnp.pad(x.astype(jnp.int32).T,
                 ((0, COLS_PAD - COLS), (0, 0)))

    k1 = pl.kernel(
        _table_body,
        out_type=jax.ShapeDtypeStruct((PAIRS, LANES), jnp.float32),
        mesh=plsc.VectorSubcoreMesh(core_axis_name="c", subcore_axis_name="s"),
        scratch_types=[
            pltpu.VMEM((D, VB), jnp.float32),
            pltpu.VMEM((D, VB), jnp.float32),
            pltpu.VMEM((VB // 2, LANES), jnp.float32),
            pltpu.VMEM((VB // 2, LANES), jnp.float32),
            pltpu.VMEM((VOCAB - NBLK * VB, LANES), jnp.float32),
            pltpu.SemaphoreType.DMA,
            pltpu.SemaphoreType.DMA,
            pltpu.SemaphoreType.DMA,
            pltpu.SemaphoreType.DMA,
        ],
        compiler_params=pltpu.CompilerParams(
            use_tc_tiling_on_sc=True, needs_layout_passes=False),
    )
    table = k1(emb_t, tail_p)

    k2 = pl.kernel(
        _gather_body,
        out_type=jax.ShapeDtypeStruct((COLS, D, ROWS), jnp.float32),
        mesh=plsc.VectorSubcoreMesh(core_axis_name="c", subcore_axis_name="s"),
        scratch_types=[
            pltpu.VMEM((COLS_PAD, SB), jnp.int32),
            pltpu.VMEM((COLS_PAD, SB), jnp.int32),
            pltpu.VMEM((SB, LANES), jnp.float32),
            pltpu.VMEM((SB, LANES), jnp.float32),
            pltpu.VMEM((D, SB), jnp.float32),
            pltpu.VMEM((D, SB), jnp.float32),
            pltpu.SemaphoreType.DMA,
            pltpu.SemaphoreType.DMA,
            pltpu.SemaphoreType.DMA,
            pltpu.SemaphoreType.DMA,
        ],
        compiler_params=pltpu.CompilerParams(
            use_tc_tiling_on_sc=True, needs_layout_passes=False),
    )
    out3 = k2(table, xt)
    return out3.transpose(2, 0, 1)


# speedup vs baseline: 1.0890x; 1.0258x over previous
import math

import jax
import jax.numpy as jnp
from jax import lax
from jax.experimental import pallas as pl
from jax.experimental.pallas import tpu as pltpu
from jax.experimental.pallas import tpu_sc as plsc

VOCAB = 1000000
D = 64
ROWS = 4096
COLS = 50
NC = 2
NS = 16
NW = NC * NS
SCALE = math.sqrt(D)

VB = 128
LANES = 128
NBLK = VOCAB // VB
BASE_BLK = NBLK // NW
EXTRA = NBLK - BASE_BLK * NW
HALF = BASE_BLK // 2
PAIRS = NBLK * (VB // 2) + D
assert VOCAB - NBLK * VB == D

SB = ROWS // NW
COLS_PAD = 56


def _table_body(emb_t, tail_p, table, in0, in1, out0, out1, tbuf,
                isem0, isem1, osem0, osem1):
    w = lax.axis_index("s") * NC + lax.axis_index("c")
    start = w * BASE_BLK + jnp.minimum(w, EXTRA)
    ins = (in0, in1)
    outs = (out0, out1)
    isems = (isem0, isem1)
    osems = (osem0, osem1)

    def src(b):
        return emb_t.at[:, pl.ds(pl.multiple_of(b * VB, VB), VB)]

    def dst(b):
        return table.at[pl.ds(pl.multiple_of(b * (VB // 2), VB // 2),
                              VB // 2), :]

    def transpose_scale(ib, ob):
        iot = lax.iota(jnp.int32, 16)
        for jc in range(D // 16):
            jv = iot + jc * 16

            @plsc.parallel_loop(0, (VB // 16) * 16, unroll=8)
            def _(i):
                vv = ((iot + i) & 15) + ((i >> 4) << 4)
                vals = plsc.load_gather(ib, [jv, vv])
                plsc.store_scatter(
                    ob, [vv & 63, (vv & 64) + jv], vals)

    pltpu.async_copy(src(start), ins[0], isems[0])
    pltpu.async_copy(src(start + 1), ins[1], isems[1])

    def step(i, carry):
        for par in range(2):
            b = start + 2 * i + par
            pltpu.make_async_copy(src(b), ins[par], isems[par]).wait()

            @pl.when(i > 0)
            def _():
                pltpu.make_async_copy(
                    dst(b - 2), outs[par], osems[par]).wait()
            transpose_scale(ins[par], outs[par])
            pltpu.async_copy(outs[par], dst(b), osems[par])

            @pl.when(2 * i + par + 2 < BASE_BLK)
            def _():
                pltpu.async_copy(src(b + 2), ins[par], isems[par])
        return carry

    lax.fori_loop(0, HALF, step, 0)
    pltpu.make_async_copy(dst(start + BASE_BLK - 2), outs[0], osems[0]).wait()
    pltpu.make_async_copy(dst(start + BASE_BLK - 1), outs[1], osems[1]).wait()

    @pl.when(w < EXTRA)
    def _():
        b = start + BASE_BLK
        pltpu.async_copy(src(b), ins[0], isems[0]).wait()
        transpose_scale(ins[0], outs[0])
        pltpu.async_copy(outs[0], dst(b), osems[0]).wait()

    @pl.when(w == NW - 1)
    def _():
        pltpu.async_copy(tail_p, tbuf, isems[0]).wait()
        pltpu.async_copy(
            tbuf, table.at[pl.ds(NBLK * (VB // 2), VOCAB - NBLK * VB), :],
            osems[0]
        ).wait()


def _gather_body(table, xt, out, xv, xp, g0, g1, o0, o1,
                 gsem0, gsem1, osem0, osem1):
    w = lax.axis_index("s") * NC + lax.axis_index("c")
    soff = pl.multiple_of(w * SB, SB)
    gs = (g0, g1)
    os_ = (o0, o1)
    gsems = (gsem0, gsem1)
    osems = (osem0, osem1)
    iot = lax.iota(jnp.int32, 16)

    pltpu.sync_copy(xt.at[:, pl.ds(soff, SB)], xv)

    @plsc.parallel_loop(0, COLS_PAD * (SB // 16), unroll=8)
    def _(i):
        r = i >> 3
        c0 = (i & 7) * 16
        v = xv[r, pl.ds(c0, 16)]
        xp[r, pl.ds(c0, 16)] = ((v >> 1) & ~63) | (v & 63)

    def gather(t, par):
        return pltpu.async_copy(table.at[xp.at[t]], gs[par], gsems[par])

    def wait_gather(t, par):
        pltpu.make_async_copy(table.at[xp.at[t]], gs[par], gsems[par]).wait()

    def put(t, par):
        return pltpu.async_copy(
            os_[par], out.at[t, :, pl.ds(soff, SB)], osems[par])

    def wait_put(t, par):
        pltpu.make_async_copy(
            os_[par], out.at[t, :, pl.ds(soff, SB)], osems[par]).wait()

    def transpose(t, g, o):
        for sb in range(SB // 16):
            s0 = sb * 16
            parv = xv[t, pl.ds(s0, 16)] & 64

            @plsc.parallel_loop(0, (D // 16) * 16, unroll=8)
            def _(i):
                d = i & 15
                jbase = i >> 4
                sv = iot + s0
                jv = ((iot + d) & 15) + jbase * 16
                vals = plsc.load_gather(g, [sv, parv + jv])
                plsc.store_scatter(o, [jv, sv], vals * SCALE)

    gather(0, 0)
    gather(1, 1)

    def step(i, carry):
        for par in range(2):
            t = 2 * i + par
            wait_gather(t, par)

            @pl.when(i > 0)
            def _():
                wait_put(t - 2, par)
            transpose(t, gs[par], os_[par])
            put(t, par)

            @pl.when(t + 2 < COLS)
            def _():
                gather(t + 2, par)
        return carry

    lax.fori_loop(0, COLS // 2, step, 0)
    wait_put(COLS - 2, 0)
    wait_put(COLS - 1, 1)


@jax.jit
def kernel(x, embedding):
    emb_t = embedding.T
    tail_p = jnp.pad(embedding[NBLK * VB:],
                     ((0, 0), (0, LANES - D)))
    xt = jnp.pad(x.astype(jnp.int32).T,
                 ((0, COLS_PAD - COLS), (0, 0)))

    k1 = pl.kernel(
        _table_body,
        out_type=jax.ShapeDtypeStruct((PAIRS, LANES), jnp.float32),
        mesh=plsc.VectorSubcoreMesh(core_axis_name="c", subcore_axis_name="s"),
        scratch_types=[
            pltpu.VMEM((D, VB), jnp.float32),
            pltpu.VMEM((D, VB), jnp.float32),
            pltpu.VMEM((VB // 2, LANES), jnp.float32),
            pltpu.VMEM((VB // 2, LANES), jnp.float32),
            pltpu.VMEM((VOCAB - NBLK * VB, LANES), jnp.float32),
            pltpu.SemaphoreType.DMA,
            pltpu.SemaphoreType.DMA,
            pltpu.SemaphoreType.DMA,
            pltpu.SemaphoreType.DMA,
        ],
        compiler_params=pltpu.CompilerParams(
            use_tc_tiling_on_sc=True, needs_layout_passes=False),
    )
    table = k1(emb_t, tail_p)

    k2 = pl.kernel(
        _gather_body,
        out_type=jax.ShapeDtypeStruct((COLS, D, ROWS), jnp.float32),
        mesh=plsc.VectorSubcoreMesh(core_axis_name="c", subcore_axis_name="s"),
        scratch_types=[
            pltpu.VMEM((COLS_PAD, SB), jnp.int32),
            pltpu.VMEM((COLS_PAD, SB), jnp.int32),
            pltpu.VMEM((SB, LANES), jnp.float32),
            pltpu.VMEM((SB, LANES), jnp.float32),
            pltpu.VMEM((D, SB), jnp.float32),
            pltpu.VMEM((D, SB), jnp.float32),
            pltpu.SemaphoreType.DMA,
            pltpu.SemaphoreType.DMA,
            pltpu.SemaphoreType.DMA,
            pltpu.SemaphoreType.DMA,
        ],
        compiler_params=pltpu.CompilerParams(
            use_tc_tiling_on_sc=True, needs_layout_passes=False),
    )
    out3 = k2(table, xt)
    return out3.transpose(2, 0, 1)
